# baseline (speedup 1.0000x reference)
import functools

import jax
import jax.numpy as jnp
from jax import lax
from jax.experimental import pallas as pl
from jax.experimental.pallas import tpu as pltpu
from jax.experimental.pallas import tpu_sc as plsc

BATCH = 16384
LATENT = 32
DESC = 1024

_NC = 2
_NS = 16
_NW = _NC * _NS
_BPW = BATCH // _NW
_L = 16


def _mf_scores(user_id, item_id, user_table, item_table):
    mesh = plsc.VectorSubcoreMesh(core_axis_name="c", subcore_axis_name="s")

    @functools.partial(
        pl.kernel,
        mesh=mesh,
        compiler_params=pltpu.CompilerParams(
            needs_layout_passes=False, use_tc_tiling_on_sc=False),
        out_type=jax.ShapeDtypeStruct((BATCH,), jnp.float32),
        scratch_types=[
            pltpu.VMEM((_BPW,), jnp.int32),
            pltpu.VMEM((_BPW,), jnp.int32),
            pltpu.VMEM((_BPW, LATENT), jnp.float32),
            pltpu.VMEM((_BPW, LATENT), jnp.float32),
            pltpu.VMEM((_BPW,), jnp.float32),
            pltpu.SemaphoreType.DMA,
            pltpu.SemaphoreType.DMA,
        ],
    )
    def k(uid_hbm, iid_hbm, ut_hbm, it_hbm, out_hbm,
          uidx_v, iidx_v, u_v, i_v, s_v, sem_u, sem_i):
        wid = lax.axis_index("s") * _NC + lax.axis_index("c")
        base = wid * _BPW
        pltpu.sync_copy(uid_hbm.at[pl.ds(base, _BPW)], uidx_v)
        pltpu.sync_copy(iid_hbm.at[pl.ds(base, _BPW)], iidx_v)
        cu = pltpu.async_copy(ut_hbm.at[uidx_v], u_v, sem_u)
        ci = pltpu.async_copy(it_hbm.at[iidx_v], i_v, sem_i)
        cu.wait()
        ci.wait()

        lane = lax.iota(jnp.int32, _L)

        def body(g, carry):
            acc = jnp.zeros((_L,), jnp.float32)
            for j in range(_L):
                r = g * _L + j
                p = (u_v[r, pl.ds(0, _L)] * i_v[r, pl.ds(0, _L)]
                     + u_v[r, pl.ds(_L, _L)] * i_v[r, pl.ds(_L, _L)])
                s = lax.reduce_sum_p.bind(p, axes=(0,))
                acc = jnp.where(lane == j, s, acc)
            s_v[pl.ds(g * _L, _L)] = acc
            return carry

        lax.fori_loop(0, _BPW // _L, body, 0)
        pltpu.sync_copy(s_v, out_hbm.at[pl.ds(base, _BPW)])

    return k(user_id, item_id, user_table, item_table)


def _desc_scores(item_descriptive, W_desc, b_desc):
    blk = 1024
    grid = BATCH // blk

    def body(x_ref, w_ref, b_ref, o_ref):
        s = jnp.sum(x_ref[...] * w_ref[...], axis=1)
        o_ref[...] = s + b_ref[0, 0]

    out = pl.pallas_call(
        body,
        grid=(grid,),
        in_specs=[
            pl.BlockSpec((blk, DESC), lambda i: (i, 0)),
            pl.BlockSpec((1, DESC), lambda i: (0, 0)),
            pl.BlockSpec((1, 1), lambda i: (0, 0)),
        ],
        out_specs=pl.BlockSpec((blk,), lambda i: (i,)),
        out_shape=jax.ShapeDtypeStruct((BATCH,), jnp.float32),
    )(item_descriptive, W_desc, b_desc.reshape(1, 1))
    return out


def kernel(user_id, item_id, item_descriptive, user_table, item_table,
           W_desc, b_desc):
    mf = _mf_scores(user_id.astype(jnp.int32), item_id.astype(jnp.int32),
                    user_table, item_table)
    de = _desc_scores(item_descriptive, W_desc, b_desc)
    return mf + de


# speedup vs baseline: 3.5185x; 3.5185x over previous
import functools

import jax
import jax.numpy as jnp
from jax import lax
from jax.experimental import pallas as pl
from jax.experimental.pallas import tpu as pltpu
from jax.experimental.pallas import tpu_sc as plsc

BATCH = 16384
LATENT = 32
DESC = 1024
NROWS = 1000000

_NC = 2
_NS = 16
_NW = _NC * _NS
_BPW = BATCH // _NW
_L = 16
_G = 4
_NG = _BPW // _G
_MAXCOL = NROWS - 128


def _col_of(uid):
    return jnp.minimum((uid // 128) * 128, _MAXCOL)


def _mf_scores(user_id, item_id, user_table_t, item_table_t):
    mesh = plsc.VectorSubcoreMesh(core_axis_name="c", subcore_axis_name="s")

    @functools.partial(
        pl.kernel,
        mesh=mesh,
        compiler_params=pltpu.CompilerParams(needs_layout_passes=False),
        out_type=jax.ShapeDtypeStruct((BATCH,), jnp.float32),
        scratch_types=[
            pltpu.VMEM((_BPW + _L,), jnp.int32),
            pltpu.VMEM((_BPW + _L,), jnp.int32),
            pltpu.VMEM((2, LATENT, _G * 128), jnp.float32),
            pltpu.VMEM((2, LATENT, _G * 128), jnp.float32),
            pltpu.VMEM((_BPW,), jnp.float32),
            pltpu.SemaphoreType.DMA,
            pltpu.SemaphoreType.DMA,
        ],
    )
    def k(uid_hbm, iid_hbm, ut_hbm, it_hbm, out_hbm,
          uidx_v, iidx_v, u_v, i_v, s_v, sem_a, sem_b):
        wid = lax.axis_index("s") * _NC + lax.axis_index("c")
        base = wid * _BPW
        pltpu.sync_copy(uid_hbm.at[pl.ds(base, _BPW)],
                        uidx_v.at[pl.ds(0, _BPW)])
        pltpu.sync_copy(iid_hbm.at[pl.ds(base, _BPW)],
                        iidx_v.at[pl.ds(0, _BPW)])

        row16 = lax.iota(jnp.int32, _L)
        sems = (sem_a, sem_b)

        def fetch(g, buf, sem):
            ids_u = uidx_v[pl.ds(g * _G, _L)]
            ids_i = iidx_v[pl.ds(g * _G, _L)]
            for j in range(_G):
                uid = jnp.clip(ids_u[j], 0, NROWS - 1)
                iid = jnp.clip(ids_i[j], 0, NROWS - 1)
                ucol = pl.multiple_of(_col_of(uid), 128)
                icol = pl.multiple_of(_col_of(iid), 128)
                pltpu.async_copy(
                    ut_hbm.at[:, pl.ds(ucol, 128)],
                    u_v.at[buf, :, pl.ds(j * 128, 128)], sem)
                pltpu.async_copy(
                    it_hbm.at[:, pl.ds(icol, 128)],
                    i_v.at[buf, :, pl.ds(j * 128, 128)], sem)

        def drain(sem):
            for _ in range(2 * _G):
                pltpu.make_async_copy(
                    ut_hbm.at[:, pl.ds(0, 128)],
                    u_v.at[0, :, pl.ds(0, 128)], sem).wait()

        def compute(g, buf, q, acc):
            ids_u = uidx_v[pl.ds(g * _G, _L)]
            ids_i = iidx_v[pl.ds(g * _G, _L)]
            bufv = jnp.full((_L,), buf, jnp.int32)
            for j in range(_G):
                uid = ids_u[j]
                iid = ids_i[j]
                ulane = uid - _col_of(uid) + j * 128
                ilane = iid - _col_of(iid) + j * 128
                p = jnp.zeros((_L,), jnp.float32)
                for h in range(2):
                    rows = row16 + h * _L
                    uvec = plsc.load_gather(
                        u_v, [bufv, rows, jnp.full((_L,), ulane, jnp.int32)])
                    ivec = plsc.load_gather(
                        i_v, [bufv, rows, jnp.full((_L,), ilane, jnp.int32)])
                    p = p + uvec * ivec
                s = lax.reduce_sum_p.bind(p, axes=(0,))
                acc = jnp.where(row16 == q * _G + j, s, acc)
            return acc

        fetch(0, 0, sem_a)

        def body(quad, carry):
            acc = jnp.zeros((_L,), jnp.float32)
            for q in range(4):
                g = quad * 4 + q
                buf = q % 2
                nbuf = (q + 1) % 2

                @pl.when(g + 1 < _NG)
                def _():
                    fetch(g + 1, nbuf, sems[(q + 1) % 2])

                drain(sems[q % 2])
                acc = compute(g, buf, q, acc)
            s_v[pl.ds(quad * _L, _L)] = acc
            return carry

        lax.fori_loop(0, _NG // 4, body, 0)
        pltpu.sync_copy(s_v, out_hbm.at[pl.ds(base, _BPW)])

    return k(user_id, item_id, user_table_t, item_table_t)


def _desc_scores(item_descriptive, W_desc, b_desc):
    blk = 1024
    grid = BATCH // blk

    def body(x_ref, w_ref, b_ref, o_ref):
        s = jnp.sum(x_ref[...] * w_ref[...], axis=1)
        o_ref[...] = s + b_ref[0, 0]

    out = pl.pallas_call(
        body,
        grid=(grid,),
        in_specs=[
            pl.BlockSpec((blk, DESC), lambda i: (i, 0)),
            pl.BlockSpec((1, DESC), lambda i: (0, 0)),
            pl.BlockSpec((1, 1), lambda i: (0, 0)),
        ],
        out_specs=pl.BlockSpec((blk,), lambda i: (i,)),
        out_shape=jax.ShapeDtypeStruct((BATCH,), jnp.float32),
    )(item_descriptive, W_desc, b_desc.reshape(1, 1))
    return out


def kernel(user_id, item_id, item_descriptive, user_table, item_table,
           W_desc, b_desc):
    mf = _mf_scores(user_id.astype(jnp.int32), item_id.astype(jnp.int32),
                    user_table.T, item_table.T)
    de = _desc_scores(item_descriptive, W_desc, b_desc)
    return mf + de
